# baseline (speedup 1.0000x reference)
import jax
import jax.numpy as jnp
from jax.experimental import pallas as pl

D = 512
K = 1024
Q = 12
B = 256


def _cc_kernel(cb_ref, cc_ref):
    cb = cb_ref[...]
    cc_ref[...] = jnp.sum(cb * cb, axis=-1)


def _rvq_kernel(x_ref, cb_ref, cc_ref, out_ref):
    r = x_ref[...]
    iota = jax.lax.broadcasted_iota(jnp.int32, (r.shape[0], K), 1)
    cols = []
    for q in range(Q):
        cb = cb_ref[q]
        rr = jnp.sum(r * r, axis=1, keepdims=True)
        s = jax.lax.dot_general(
            r.astype(jnp.bfloat16), cb.astype(jnp.bfloat16),
            (((1,), (1,)), ((), ())),
            preferred_element_type=jnp.float32)
        dist = rr - 2.0 * s + cc_ref[q][None, :]
        mn = jnp.min(dist, axis=1, keepdims=True)
        idx = jnp.min(jnp.where(dist == mn, iota, K), axis=1)
        oh = (iota == idx[:, None]).astype(jnp.float32)
        quant = jax.lax.dot_general(
            oh, cb, (((1,), (0,)), ((), ())),
            precision=jax.lax.Precision.HIGHEST,
            preferred_element_type=jnp.float32)
        r = r - quant
        cols.append(idx)
    out_ref[...] = jnp.stack(cols, axis=-1)


def kernel(embedding, codebooks):
    n = embedding.shape[0]
    cc = pl.pallas_call(
        _cc_kernel,
        out_shape=jax.ShapeDtypeStruct((Q, K), jnp.float32),
    )(codebooks)
    out = pl.pallas_call(
        _rvq_kernel,
        grid=(n // B,),
        in_specs=[
            pl.BlockSpec((B, D), lambda i: (i, 0)),
            pl.BlockSpec((Q, K, D), lambda i: (0, 0, 0)),
            pl.BlockSpec((Q, K), lambda i: (0, 0)),
        ],
        out_specs=pl.BlockSpec((B, Q), lambda i: (i, 0)),
        out_shape=jax.ShapeDtypeStruct((n, Q), jnp.int32),
    )(embedding, codebooks, cc)
    return out[:, :, None]


# speedup vs baseline: 1.8136x; 1.8136x over previous
import jax
import jax.numpy as jnp
from jax.experimental import pallas as pl
from jax.experimental.pallas import tpu as pltpu

D = 512
K = 1024
Q = 12
B = 512


def _prep_kernel(cb_ref, cc_ref, hi_ref, mid_ref, lo_ref):
    cb = cb_ref[...]
    cc_ref[...] = jnp.sum(cb * cb, axis=-1, keepdims=True).transpose(0, 2, 1)
    hi = cb.astype(jnp.bfloat16)
    e1 = cb - hi.astype(jnp.float32)
    mid = e1.astype(jnp.bfloat16)
    lo = (e1 - mid.astype(jnp.float32)).astype(jnp.bfloat16)
    hi_ref[...] = hi
    mid_ref[...] = mid
    lo_ref[...] = lo


def _rvq_kernel(x_ref, hi_ref, mid_ref, lo_ref, cc_ref, out_ref):
    r = x_ref[...]
    nrows = r.shape[0]
    iota = jax.lax.broadcasted_iota(jnp.int32, (nrows, K), 1)
    cols = []
    for q in range(Q):
        hi = hi_ref[q]
        rr = jnp.sum(r * r, axis=1, keepdims=True)
        s2 = jax.lax.dot_general(
            (r + r).astype(jnp.bfloat16), hi,
            (((1,), (1,)), ((), ())),
            preferred_element_type=jnp.float32)
        dist = rr - s2 + cc_ref[q]
        mn = jnp.min(dist, axis=1, keepdims=True)
        idx = jnp.min(jnp.where(dist == mn, iota, K), axis=1)
        cols.append(idx)
        if q < Q - 1:
            oh = (iota == idx[:, None]).astype(jnp.bfloat16)
            dn = (((1,), (0,)), ((), ()))
            quant = (
                jax.lax.dot_general(oh, hi, dn,
                                    preferred_element_type=jnp.float32)
                + jax.lax.dot_general(oh, mid_ref[q], dn,
                                      preferred_element_type=jnp.float32)
                + jax.lax.dot_general(oh, lo_ref[q], dn,
                                      preferred_element_type=jnp.float32)
            )
            r = r - quant
    out_ref[...] = jnp.stack(cols, axis=-1)


def kernel(embedding, codebooks):
    n = embedding.shape[0]
    bf = jnp.bfloat16
    cc, hi, mid, lo = pl.pallas_call(
        _prep_kernel,
        grid=(Q,),
        in_specs=[pl.BlockSpec((1, K, D), lambda q: (q, 0, 0))],
        out_specs=[
            pl.BlockSpec((1, 1, K), lambda q: (q, 0, 0)),
            pl.BlockSpec((1, K, D), lambda q: (q, 0, 0)),
            pl.BlockSpec((1, K, D), lambda q: (q, 0, 0)),
            pl.BlockSpec((1, K, D), lambda q: (q, 0, 0)),
        ],
        out_shape=[
            jax.ShapeDtypeStruct((Q, 1, K), jnp.float32),
            jax.ShapeDtypeStruct((Q, K, D), bf),
            jax.ShapeDtypeStruct((Q, K, D), bf),
            jax.ShapeDtypeStruct((Q, K, D), bf),
        ],
    )(codebooks)
    out = pl.pallas_call(
        _rvq_kernel,
        grid=(n // B,),
        in_specs=[
            pl.BlockSpec((B, D), lambda i: (i, 0)),
            pl.BlockSpec((Q, K, D), lambda i: (0, 0, 0)),
            pl.BlockSpec((Q, K, D), lambda i: (0, 0, 0)),
            pl.BlockSpec((Q, K, D), lambda i: (0, 0, 0)),
            pl.BlockSpec((Q, 1, K), lambda i: (0, 0, 0)),
        ],
        out_specs=pl.BlockSpec((B, Q), lambda i: (i, 0)),
        out_shape=jax.ShapeDtypeStruct((n, Q), jnp.int32),
        compiler_params=pltpu.CompilerParams(
            dimension_semantics=("parallel",)),
    )(embedding, hi, mid, lo, cc)
    return out[:, :, None]
